# baseline (speedup 1.0000x reference)
import functools

import jax
import jax.numpy as jnp
from jax import lax
from jax.experimental import pallas as pl
from jax.experimental.pallas import tpu as pltpu
from jax.experimental.pallas import tpu_sc as plsc

_VOCAB = 1000000
_EMBD = 32
_BATCH = 16384
_NNEG = 5
_W = _EMBD // 2

_NC = 2
_NS = 16
_L = 16
_NW = _NC * _NS
_BPW = _BATCH // _NW
_CH = 128
_NCHUNK = _BPW // _CH
_GPC = _CH // _L


def _unpack2(w):
    lo = plsc.bitcast(w << 16, jnp.float32)
    hi = plsc.bitcast(w & jnp.int32(-65536), jnp.float32)
    return lo, hi


def _sc_scores_body(u_row_hbm, p_row_hbm, n_row_hbm,
                    U_hbm, V_hbm,
                    pos_out, neg_out,
                    urowi, prowi, nrowi,
                    urows, prows, nrows, psc, nsc, sem):
    wid = lax.axis_index("s") * _NC + lax.axis_index("c")
    base = wid * _BPW

    pltpu.sync_copy(u_row_hbm.at[pl.ds(base, _BPW)], urowi)
    pltpu.sync_copy(p_row_hbm.at[pl.ds(base, _BPW)], prowi)
    pltpu.sync_copy(n_row_hbm.at[pl.ds(base * _NNEG, _BPW * _NNEG)], nrowi)

    iot = lax.iota(jnp.int32, _L)

    for c in range(_NCHUNK):
        copies = [
            pltpu.async_copy(
                U_hbm.at[urowi.at[pl.ds(c * _CH, _CH)]], urows, sem),
            pltpu.async_copy(
                V_hbm.at[prowi.at[pl.ds(c * _CH, _CH)]], prows, sem),
        ]
        for j in range(_NNEG):
            copies.append(pltpu.async_copy(
                V_hbm.at[nrowi.at[pl.ds((c * _NNEG + j) * _CH, _CH)]],
                nrows.at[pl.ds(j * _CH, _CH)], sem))
        for cp in copies:
            cp.wait()

        def group(g, carry):
            rb = g * _L + iot
            pos_acc = jnp.zeros((_L,), jnp.float32)
            neg_acc = jnp.zeros((_L,), jnp.float32)
            for d2 in range(_W):
                dcol = jnp.full((_L,), d2, jnp.int32)
                ulo, uhi = _unpack2(plsc.load_gather(urows, [rb, dcol]))
                plo, phi = _unpack2(plsc.load_gather(prows, [rb, dcol]))
                nlo, nhi = _unpack2(
                    plsc.load_gather(nrows, [rb * _NNEG, dcol]))
                for n in range(1, _NNEG):
                    l2, h2 = _unpack2(
                        plsc.load_gather(nrows, [rb * _NNEG + n, dcol]))
                    nlo = nlo + l2
                    nhi = nhi + h2
                pos_acc = pos_acc + ulo * plo + uhi * phi
                neg_acc = neg_acc + ulo * nlo + uhi * nhi
            psc[pl.ds(c * _CH + g * _L, _L)] = pos_acc
            nsc[pl.ds(c * _CH + g * _L, _L)] = neg_acc
            return carry

        lax.fori_loop(0, _GPC, group, 0)

    pltpu.sync_copy(psc, pos_out.at[pl.ds(base, _BPW)])
    pltpu.sync_copy(nsc, neg_out.at[pl.ds(base, _BPW)])


_sc_scores = functools.partial(
    pl.kernel,
    out_type=[jax.ShapeDtypeStruct((_BATCH,), jnp.float32),
              jax.ShapeDtypeStruct((_BATCH,), jnp.float32)],
    mesh=plsc.VectorSubcoreMesh(core_axis_name="c", subcore_axis_name="s",
                                num_cores=_NC, num_subcores=_NS),
    compiler_params=pltpu.CompilerParams(needs_layout_passes=False,
                                         use_tc_tiling_on_sc=False),
    scratch_types=[
        pltpu.VMEM((_BPW,), jnp.int32),
        pltpu.VMEM((_BPW,), jnp.int32),
        pltpu.VMEM((_BPW * _NNEG,), jnp.int32),
        pltpu.VMEM((_CH, _W), jnp.int32),
        pltpu.VMEM((_CH, _W), jnp.int32),
        pltpu.VMEM((_CH * _NNEG, _W), jnp.int32),
        pltpu.VMEM((_BPW,), jnp.float32),
        pltpu.VMEM((_BPW,), jnp.float32),
        pltpu.SemaphoreType.DMA,
    ],
)(_sc_scores_body)


_WV = 8192
_PG = -(-_VOCAB // _WV)


def _tc_pack_body(in_ref, out_ref):
    x = in_ref[...]
    u = lax.bitcast_convert_type(x, jnp.uint32)
    r = (u + jnp.uint32(0x7FFF) + ((u >> 16) & jnp.uint32(1))) >> 16
    w = r[0:_W, :] | (r[_W:_EMBD, :] << 16)
    out_ref[...] = lax.bitcast_convert_type(w, jnp.int32)


def _pack_table(T):
    packed = pl.pallas_call(
        _tc_pack_body,
        grid=(_PG,),
        in_specs=[pl.BlockSpec((_EMBD, _WV), lambda i: (0, i))],
        out_specs=pl.BlockSpec((_W, _WV), lambda i: (0, i)),
        out_shape=jax.ShapeDtypeStruct((_W, _PG * _WV), jnp.int32),
    )(T.T)
    return packed.T


def _tc_final_body(pos_ref, neg_ref, out_ref):
    pos = pos_ref[...]
    neg = neg_ref[...]
    total = (jnp.sum(jax.nn.log_sigmoid(pos))
             + jnp.sum(jax.nn.log_sigmoid(-neg)))
    out_ref[0, 0] = -total / _BATCH


def _tc_final(pos2d, neg2d):
    return pl.pallas_call(
        _tc_final_body,
        out_shape=jax.ShapeDtypeStruct((1, 1), jnp.float32),
        out_specs=pl.BlockSpec(memory_space=pltpu.SMEM),
    )(pos2d, neg2d)


def kernel(u_pos, v_pos, v_neg, U, V):
    u = u_pos.astype(jnp.int32)
    p = v_pos.astype(jnp.int32)
    n = v_neg.astype(jnp.int32).reshape(_BATCH * _NNEG)
    pos, neg = _sc_scores(u, p, n, _pack_table(U), _pack_table(V))
    res = _tc_final(pos.reshape(_BATCH // 128, 128),
                    neg.reshape(_BATCH // 128, 128))
    return res[0, 0]


# speedup vs baseline: 2.5022x; 2.5022x over previous
import functools

import jax
import jax.numpy as jnp
from jax import lax
from jax.experimental import pallas as pl
from jax.experimental.pallas import tpu as pltpu
from jax.experimental.pallas import tpu_sc as plsc

_VOCAB = 1000000
_EMBD = 32
_BATCH = 16384
_NNEG = 5
_W = _EMBD // 2

_NC = 2
_NS = 16
_L = 16
_NW = _NC * _NS
_BPW = _BATCH // _NW
_CH = 128
_NCHUNK = _BPW // _CH
_GPC = _CH // _L

_WV = 8192
_PG = -(-_VOCAB // _WV)
_VPAD = _PG * _WV


def _unpack2(w):
    lo = plsc.bitcast(w << 16, jnp.float32)
    hi = plsc.bitcast(w & jnp.int32(-65536), jnp.float32)
    return lo, hi


def _sc_scores_body(u_row_hbm, p_row_hbm, n_row_hbm,
                    U_hbm, V_hbm,
                    pos_out, neg_out,
                    urowi, prowi, nrowi,
                    urows, prows, nrows, psc, nsc, sem):
    wid = lax.axis_index("s") * _NC + lax.axis_index("c")
    base = wid * _BPW

    pltpu.sync_copy(u_row_hbm.at[pl.ds(base, _BPW)], urowi)
    pltpu.sync_copy(p_row_hbm.at[pl.ds(base, _BPW)], prowi)
    pltpu.sync_copy(n_row_hbm.at[pl.ds(base * _NNEG, _BPW * _NNEG)], nrowi)

    iot = lax.iota(jnp.int32, _L)

    for c in range(_NCHUNK):
        copies = [
            pltpu.async_copy(
                U_hbm.at[urowi.at[pl.ds(c * _CH, _CH)]], urows, sem),
            pltpu.async_copy(
                V_hbm.at[prowi.at[pl.ds(c * _CH, _CH)]], prows, sem),
        ]
        for j in range(_NNEG):
            copies.append(pltpu.async_copy(
                V_hbm.at[nrowi.at[pl.ds((c * _NNEG + j) * _CH, _CH)]],
                nrows.at[pl.ds(j * _CH, _CH)], sem))
        for cp in copies:
            cp.wait()

        def group(g, carry):
            rb = g * _L + iot
            pos_acc = jnp.zeros((_L,), jnp.float32)
            neg_acc = jnp.zeros((_L,), jnp.float32)
            for d2 in range(_W):
                dcol = jnp.full((_L,), d2, jnp.int32)
                ulo, uhi = _unpack2(plsc.load_gather(urows, [rb, dcol]))
                plo, phi = _unpack2(plsc.load_gather(prows, [rb, dcol]))
                nlo, nhi = _unpack2(
                    plsc.load_gather(nrows, [rb * _NNEG, dcol]))
                for n in range(1, _NNEG):
                    l2, h2 = _unpack2(
                        plsc.load_gather(nrows, [rb * _NNEG + n, dcol]))
                    nlo = nlo + l2
                    nhi = nhi + h2
                pos_acc = pos_acc + ulo * plo + uhi * phi
                neg_acc = neg_acc + ulo * nlo + uhi * nhi
            psc[pl.ds(c * _CH + g * _L, _L)] = pos_acc
            nsc[pl.ds(c * _CH + g * _L, _L)] = neg_acc
            return carry

        lax.fori_loop(0, _GPC, group, 0)

    pltpu.sync_copy(psc, pos_out.at[pl.ds(base, _BPW)])
    pltpu.sync_copy(nsc, neg_out.at[pl.ds(base, _BPW)])


_sc_scores = functools.partial(
    pl.kernel,
    out_type=[jax.ShapeDtypeStruct((_BATCH,), jnp.float32),
              jax.ShapeDtypeStruct((_BATCH,), jnp.float32)],
    mesh=plsc.VectorSubcoreMesh(core_axis_name="c", subcore_axis_name="s",
                                num_cores=_NC, num_subcores=_NS),
    compiler_params=pltpu.CompilerParams(needs_layout_passes=False),
    scratch_types=[
        pltpu.VMEM((_BPW,), jnp.int32),
        pltpu.VMEM((_BPW,), jnp.int32),
        pltpu.VMEM((_BPW * _NNEG,), jnp.int32),
        pltpu.VMEM((_CH, 128), jnp.int32),
        pltpu.VMEM((_CH, 128), jnp.int32),
        pltpu.VMEM((_CH * _NNEG, 128), jnp.int32),
        pltpu.VMEM((_BPW,), jnp.float32),
        pltpu.VMEM((_BPW,), jnp.float32),
        pltpu.SemaphoreType.DMA,
    ],
)(_sc_scores_body)


def _tc_pack_body(in_ref, out_ref):
    x = in_ref[...]
    u = lax.bitcast_convert_type(x, jnp.uint32)
    r = (u + jnp.uint32(0x7FFF) + ((u >> 16) & jnp.uint32(1))) >> 16
    w = r[0:_W, :] | (r[_W:_EMBD, :] << 16)
    t = lax.bitcast_convert_type(w, jnp.int32).T
    out_ref[:, pl.ds(0, _W)] = t


def _pack_table(T):
    return pl.pallas_call(
        _tc_pack_body,
        grid=(_PG,),
        in_specs=[pl.BlockSpec((_EMBD, _WV), lambda i: (0, i))],
        out_specs=pl.BlockSpec((_WV, 128), lambda i: (i, 0)),
        out_shape=jax.ShapeDtypeStruct((_VPAD, 128), jnp.int32),
    )(T.T)


def _tc_final_body(pos_ref, neg_ref, out_ref):
    pos = pos_ref[...]
    neg = neg_ref[...]
    total = (jnp.sum(jax.nn.log_sigmoid(pos))
             + jnp.sum(jax.nn.log_sigmoid(-neg)))
    out_ref[0, 0] = -total / _BATCH


def _tc_final(pos2d, neg2d):
    return pl.pallas_call(
        _tc_final_body,
        out_shape=jax.ShapeDtypeStruct((1, 1), jnp.float32),
        out_specs=pl.BlockSpec(memory_space=pltpu.SMEM),
    )(pos2d, neg2d)


def kernel(u_pos, v_pos, v_neg, U, V):
    u = u_pos.astype(jnp.int32)
    p = v_pos.astype(jnp.int32)
    n = v_neg.astype(jnp.int32).reshape(_BATCH * _NNEG)
    pos, neg = _sc_scores(u, p, n, _pack_table(U), _pack_table(V))
    res = _tc_final(pos.reshape(_BATCH // 128, 128),
                    neg.reshape(_BATCH // 128, 128))
    return res[0, 0]
